# baseline (speedup 1.0000x reference)
import functools

import jax
import jax.numpy as jnp
from jax import lax
from jax.experimental import pallas as pl
from jax.experimental.pallas import tpu as pltpu
from jax.experimental.pallas import tpu_sc as plsc

K = 256
N_ROWS = 128
N_COLS = 2048
NW = 32
ROWS_PER_W = N_ROWS // NW
NCHUNK = N_COLS // 16
INT_MIN32 = -2147483648


def _srl(a, n):
    return lax.shift_right_logical(a, jnp.full(a.shape, n, a.dtype))


def _sra(a, n):
    return lax.shift_right_arithmetic(a, jnp.full(a.shape, n, a.dtype))


def _sc_select(x_hbm, tau_hbm, x_v0, x_v1, x_v2, x_v3, keys_x, ckeys,
               hist_v, hist_m, o_v0, o_v1, o_v2, o_v3, sem_in, sem_out):
    wid = lax.axis_index("s") * 2 + lax.axis_index("c")
    base = wid * ROWS_PER_W
    xrows = [x_v0, x_v1, x_v2, x_v3]
    orows = [o_v0, o_v1, o_v2, o_v3]
    copies = [pltpu.async_copy(x_hbm.at[base + r], xrows[r], sem_in)
              for r in range(ROWS_PER_W)]
    for c in copies:
        c.wait()

    iota = lax.iota(jnp.int32, 16)
    lane16 = iota * 16
    ones = jnp.ones((16,), jnp.float32)
    zvec = jnp.zeros((16,), jnp.float32)
    zivec = jnp.zeros((16,), jnp.int32)

    def dyng(vec, lane_v):
        return vec.at[lane_v].get(mode="promise_in_bounds")

    def suffix(vec):
        return lax.rev(jnp.cumsum(lax.rev(vec, (0,))), (0,))

    def scan_hist(rem_v):
        totv = zvec
        for c in range(16):
            totv = totv + plsc.load_gather(hist_v, [lane16 + c])
        sstv = suffix(totv)
        pcv = zivec + plsc.all_reduce_population_count(sstv >= rem_v)
        jstar = pcv - 1
        tot_j = dyng(totv, jstar)
        sst_j = dyng(sstv, jstar)
        above_vecs = sst_j - tot_j
        h = plsc.load_gather(hist_v, [jstar * 16 + iota])
        ssv = suffix(h)
        pc2 = zivec + plsc.all_reduce_population_count(
            (above_vecs + ssv) >= rem_v)
        lstar = pc2 - 1
        ss_l = dyng(ssv, lstar)
        h_l = dyng(h, lstar)
        bstar = jstar * 16 + lstar
        above = above_vecs + ss_l - h_l
        return bstar, above, h_l

    def zero_hist():
        for j in range(16):
            hist_v[pl.ds(j * 16, 16)] = zvec

    UNROLL = 8
    out_copies = []
    for r in range(ROWS_PER_W):
        xrow = xrows[r]
        rem = zvec + jnp.float32(K)

        def zero_m(i, c):
            for u in range(UNROLL):
                hist_m[pl.ds((i * UNROLL + u) * 16, 16)] = zvec
            return c
        lax.fori_loop(0, 256 // UNROLL, zero_m, jnp.int32(0))

        def body0(i, c):
            for u in range(UNROLL):
                xv = xrow[pl.ds(i * (16 * UNROLL) + u * 16, 16)]
                ib = lax.bitcast_convert_type(xv, jnp.int32)
                sgn = _sra(ib, 31)
                uk = ib ^ ((sgn & 0x7FFFFFFF) | INT_MIN32)
                keys_x[pl.ds(i * (16 * UNROLL) + u * 16, 16)] = uk
                slot = (_srl(uk, 20) & 0xFF0) + iota
                plsc.addupdate_scatter(hist_m, [slot], ones)
            return c
        lax.fori_loop(0, NCHUNK // UNROLL, body0, jnp.int32(0))

        for j in range(16):
            acc = zvec
            for k in range(16):
                acc = acc + plsc.load_gather(hist_m, [j * 256 + lane16 + k])
            hist_v[pl.ds(j * 16, 16)] = acc

        b0, above0, n1f = scan_hist(rem)
        rem = rem - above0

        def bodyc(i, off_v):
            for u in range(UNROLL):
                uk = keys_x[pl.ds(i * (16 * UNROLL) + u * 16, 16)]
                msk = _srl(uk, 24) == b0
                mi = jnp.where(msk, jnp.int32(1), jnp.int32(0))
                excl = jnp.cumsum(mi) - mi
                plsc.store_scatter(ckeys, [off_v + excl], uk, mask=msk)
                off_v = off_v + plsc.all_reduce_population_count(msk)
            return off_v
        lax.fori_loop(0, NCHUNK // UNROLL, bodyc,
                      jnp.zeros((16,), jnp.int32))

        n1 = n1f.astype(jnp.int32)
        nch1 = jnp.max(_srl(n1 + 15, 4))

        b1 = b2 = b3 = None
        for l in range(1, 4):
            zero_hist()

            def bodyl(i, c, _l=l, _b1=b1, _b2=b2):
                uk = ckeys[pl.ds(i * 16, 16)]
                msk = (i * 16 + iota) < n1
                if _l >= 2:
                    msk = msk & ((_srl(uk, 16) & 0xFF) == _b1)
                if _l >= 3:
                    msk = msk & ((_srl(uk, 8) & 0xFF) == _b2)
                binv = _srl(uk, 24 - 8 * _l) & 0xFF
                plsc.addupdate_scatter(hist_v, [binv], ones, mask=msk)
                return c
            lax.fori_loop(0, nch1, bodyl, jnp.int32(0))

            bl, above_l, _hl = scan_hist(rem)
            rem = rem - above_l
            if l == 1:
                b1 = bl
            elif l == 2:
                b2 = bl
            else:
                b3 = bl

        prefix = (jnp.left_shift(b0, 24) | jnp.left_shift(b1, 16)
                  | jnp.left_shift(b2, 8) | b3)
        orows[r][...] = prefix
        out_copies.append(
            pltpu.async_copy(orows[r], tau_hbm.at[base + r], sem_out))

    for c in out_copies:
        c.wait()


def _tc_finish(x_ref, t_ref, p_ref, out_ref):
    x = x_ref[:]
    t = t_ref[:]
    p_i = p_ref[:][:, :1]
    p = jax.lax.bitcast_convert_type(p_i, jnp.uint32)

    m = jnp.max(x, axis=1, keepdims=True)

    ub = jax.lax.bitcast_convert_type(x, jnp.uint32)
    ukey = jnp.where(ub >= jnp.uint32(0x80000000), ~ub,
                     ub | jnp.uint32(0x80000000))

    ub_tau = jnp.where(p >= jnp.uint32(0x80000000),
                       p ^ jnp.uint32(0x80000000), ~p)
    tau = jax.lax.bitcast_convert_type(ub_tau, jnp.float32)

    gt = ukey > p
    c_gt = jnp.sum(gt.astype(jnp.int32), axis=1, keepdims=True)
    e = jnp.exp(x - m)
    s_above = jnp.sum(jnp.where(gt, e, 0.0), axis=1, keepdims=True)
    S = s_above + (K - c_gt).astype(jnp.float32) * jnp.exp(tau - m)

    col = jax.lax.broadcasted_iota(jnp.int32, (N_ROWS, N_COLS), 1)
    at_t = col == t
    v = jnp.sum(jnp.where(at_t, x, 0.0), axis=1, keepdims=True)
    ukey_i = jax.lax.bitcast_convert_type(ukey, jnp.int32)
    tu_i = jnp.sum(jnp.where(at_t, ukey_i, 0), axis=1, keepdims=True)
    tu = jax.lax.bitcast_convert_type(tu_i, jnp.uint32)

    eq_before = jnp.sum(((col < t) & (ukey == p)).astype(jnp.int32),
                        axis=1, keepdims=True)
    in_topk = (tu > p) | ((tu == p) & (c_gt + eq_before < K))
    inf = in_topk.astype(jnp.float32)

    contrib = v - m - jnp.log(S)
    total = jnp.sum(inf * contrib)
    count = jnp.sum(inf)
    out_ref[:, :] = jnp.full((1, 1), -(total / count), dtype=jnp.float32)


def kernel(outputs, targets):
    t32 = targets.astype(jnp.int32).reshape(N_ROWS, 1)

    sc = pl.kernel(
        _sc_select,
        out_type=jax.ShapeDtypeStruct((N_ROWS, 16), jnp.int32),
        mesh=plsc.VectorSubcoreMesh(core_axis_name="c", subcore_axis_name="s"),
        compiler_params=pltpu.CompilerParams(needs_layout_passes=False),
        scratch_types=[
            pltpu.VMEM((N_COLS,), jnp.float32),
            pltpu.VMEM((N_COLS,), jnp.float32),
            pltpu.VMEM((N_COLS,), jnp.float32),
            pltpu.VMEM((N_COLS,), jnp.float32),
            pltpu.VMEM((N_COLS,), jnp.int32),
            pltpu.VMEM((N_COLS,), jnp.int32),
            pltpu.VMEM((256,), jnp.float32),
            pltpu.VMEM((4096,), jnp.float32),
            pltpu.VMEM((16,), jnp.int32),
            pltpu.VMEM((16,), jnp.int32),
            pltpu.VMEM((16,), jnp.int32),
            pltpu.VMEM((16,), jnp.int32),
            pltpu.SemaphoreType.DMA,
            pltpu.SemaphoreType.DMA,
        ],
    )
    tau_packed = sc(outputs)

    out = pl.pallas_call(
        _tc_finish,
        out_shape=jax.ShapeDtypeStruct((1, 1), jnp.float32),
    )(outputs, t32, tau_packed)
    return out.reshape(())


# speedup vs baseline: 1.1934x; 1.1934x over previous
import functools

import jax
import jax.numpy as jnp
from jax import lax
from jax.experimental import pallas as pl
from jax.experimental.pallas import tpu as pltpu
from jax.experimental.pallas import tpu_sc as plsc

K = 256
N_ROWS = 128
N_COLS = 2048
NW = 32
ROWS_PER_W = N_ROWS // NW
NCHUNK = N_COLS // 16
INT_MIN32 = -2147483648


def _srl(a, n):
    return lax.shift_right_logical(a, jnp.full(a.shape, n, a.dtype))


def _sra(a, n):
    return lax.shift_right_arithmetic(a, jnp.full(a.shape, n, a.dtype))


def _sc_select(x_hbm, tau_hbm, x_v0, x_v1, x_v2, x_v3, keys_x, ckeys,
               hist4, o_v0, o_v1, o_v2, o_v3, sem_in, sem_out):
    wid = lax.axis_index("s") * 2 + lax.axis_index("c")
    base = wid * ROWS_PER_W
    xrows = [x_v0, x_v1, x_v2, x_v3]
    orows = [o_v0, o_v1, o_v2, o_v3]
    copies = [pltpu.async_copy(x_hbm.at[base + r], xrows[r], sem_in)
              for r in range(ROWS_PER_W)]
    for c in copies:
        c.wait()

    iota = lax.iota(jnp.int32, 16)
    lane16 = iota * 16
    ones = jnp.ones((16,), jnp.float32)
    zvec = jnp.zeros((16,), jnp.float32)
    zivec = jnp.zeros((16,), jnp.int32)

    def dyng(vec, lane_v):
        return vec.at[lane_v].get(mode="promise_in_bounds")

    def suffix(vec):
        return lax.rev(jnp.cumsum(lax.rev(vec, (0,))), (0,))

    def scan_hist(rem_v, hb):
        totv = zvec
        for c in range(16):
            totv = totv + plsc.load_gather(hist4, [hb + lane16 + c])
        sstv = suffix(totv)
        pcv = zivec + plsc.all_reduce_population_count(sstv >= rem_v)
        jstar = pcv - 1
        tot_j = dyng(totv, jstar)
        sst_j = dyng(sstv, jstar)
        above_vecs = sst_j - tot_j
        h = plsc.load_gather(hist4, [hb + jstar * 16 + iota])
        ssv = suffix(h)
        pc2 = zivec + plsc.all_reduce_population_count(
            (above_vecs + ssv) >= rem_v)
        lstar = pc2 - 1
        ss_l = dyng(ssv, lstar)
        h_l = dyng(h, lstar)
        bstar = jstar * 16 + lstar
        above = above_vecs + ss_l - h_l
        return bstar, above, h_l

    def zero_hists():
        def zb(i, c):
            for u in range(4):
                hist4[pl.ds((i * 4 + u) * 16, 16)] = zvec
            return c
        lax.fori_loop(0, 16, zb, jnp.int32(0))

    def key_of(xv):
        ib = lax.bitcast_convert_type(xv, jnp.int32)
        sgn = _sra(ib, 31)
        return ib ^ ((sgn & 0x7FFFFFFF) | INT_MIN32)

    NR = ROWS_PER_W
    U = 2

    zero_hists()

    def body0(i, c):
        for u in range(U):
            off = i * (16 * U) + u * 16
            for r in range(NR):
                uk = key_of(xrows[r][pl.ds(off, 16)])
                keys_x[pl.ds(r * N_COLS + off, 16)] = uk
                plsc.addupdate_scatter(hist4, [r * 256 + _srl(uk, 24)],
                                       ones)
        return c
    lax.fori_loop(0, NCHUNK // U, body0, jnp.int32(0))

    rems = []
    b0s = []
    n1s = []
    for r in range(NR):
        b0, above0, n1f = scan_hist(zvec + jnp.float32(K), r * 256)
        rems.append(zvec + jnp.float32(K) - above0)
        b0s.append(b0)
        n1s.append(n1f.astype(jnp.int32))

    def bodyc(i, offs):
        offs = list(offs)
        for u in range(U):
            off = i * (16 * U) + u * 16
            for r in range(NR):
                uk = keys_x[pl.ds(r * N_COLS + off, 16)]
                msk = _srl(uk, 24) == b0s[r]
                mi = jnp.where(msk, jnp.int32(1), jnp.int32(0))
                excl = jnp.cumsum(mi) - mi
                plsc.store_scatter(ckeys, [r * N_COLS + offs[r] + excl],
                                   uk, mask=msk)
                offs[r] = offs[r] + plsc.all_reduce_population_count(msk)
        return tuple(offs)
    lax.fori_loop(0, NCHUNK // U, bodyc, tuple(zivec for _ in range(NR)))

    nch1 = jnp.max(_srl(jnp.maximum(jnp.maximum(n1s[0], n1s[1]),
                                    jnp.maximum(n1s[2], n1s[3])) + 15, 4))

    b1s = b2s = b3s = None
    for l in range(1, 4):
        zero_hists()

        def bodyl(i, c, _l=l, _b1=b1s, _b2=b2s):
            for r in range(NR):
                uk = ckeys[pl.ds(r * N_COLS + i * 16, 16)]
                msk = (i * 16 + iota) < n1s[r]
                if _l >= 2:
                    msk = msk & ((_srl(uk, 16) & 0xFF) == _b1[r])
                if _l >= 3:
                    msk = msk & ((_srl(uk, 8) & 0xFF) == _b2[r])
                binv = _srl(uk, 24 - 8 * _l) & 0xFF
                plsc.addupdate_scatter(hist4, [r * 256 + binv], ones,
                                       mask=msk)
            return c
        lax.fori_loop(0, nch1, bodyl, jnp.int32(0))

        bls = []
        for r in range(NR):
            bl, above_l, _hl = scan_hist(rems[r], r * 256)
            rems[r] = rems[r] - above_l
            bls.append(bl)
        if l == 1:
            b1s = bls
        elif l == 2:
            b2s = bls
        else:
            b3s = bls

    out_copies = []
    for r in range(NR):
        prefix = (jnp.left_shift(b0s[r], 24) | jnp.left_shift(b1s[r], 16)
                  | jnp.left_shift(b2s[r], 8) | b3s[r])
        orows[r][...] = prefix
        out_copies.append(
            pltpu.async_copy(orows[r], tau_hbm.at[base + r], sem_out))
    for c in out_copies:
        c.wait()


def _tc_finish(x_ref, t_ref, p_ref, out_ref):
    x = x_ref[:]
    t = t_ref[:]
    p_i = p_ref[:][:, :1]
    p = jax.lax.bitcast_convert_type(p_i, jnp.uint32)

    m = jnp.max(x, axis=1, keepdims=True)

    ub = jax.lax.bitcast_convert_type(x, jnp.uint32)
    ukey = jnp.where(ub >= jnp.uint32(0x80000000), ~ub,
                     ub | jnp.uint32(0x80000000))

    ub_tau = jnp.where(p >= jnp.uint32(0x80000000),
                       p ^ jnp.uint32(0x80000000), ~p)
    tau = jax.lax.bitcast_convert_type(ub_tau, jnp.float32)

    gt = ukey > p
    c_gt = jnp.sum(gt.astype(jnp.int32), axis=1, keepdims=True)
    e = jnp.exp(x - m)
    s_above = jnp.sum(jnp.where(gt, e, 0.0), axis=1, keepdims=True)
    S = s_above + (K - c_gt).astype(jnp.float32) * jnp.exp(tau - m)

    col = jax.lax.broadcasted_iota(jnp.int32, (N_ROWS, N_COLS), 1)
    at_t = col == t
    v = jnp.sum(jnp.where(at_t, x, 0.0), axis=1, keepdims=True)
    ukey_i = jax.lax.bitcast_convert_type(ukey, jnp.int32)
    tu_i = jnp.sum(jnp.where(at_t, ukey_i, 0), axis=1, keepdims=True)
    tu = jax.lax.bitcast_convert_type(tu_i, jnp.uint32)

    eq_before = jnp.sum(((col < t) & (ukey == p)).astype(jnp.int32),
                        axis=1, keepdims=True)
    in_topk = (tu > p) | ((tu == p) & (c_gt + eq_before < K))
    inf = in_topk.astype(jnp.float32)

    contrib = v - m - jnp.log(S)
    total = jnp.sum(inf * contrib)
    count = jnp.sum(inf)
    out_ref[:, :] = jnp.full((1, 1), -(total / count), dtype=jnp.float32)


def kernel(outputs, targets):
    t32 = targets.astype(jnp.int32).reshape(N_ROWS, 1)

    sc = pl.kernel(
        _sc_select,
        out_type=jax.ShapeDtypeStruct((N_ROWS, 16), jnp.int32),
        mesh=plsc.VectorSubcoreMesh(core_axis_name="c", subcore_axis_name="s"),
        compiler_params=pltpu.CompilerParams(needs_layout_passes=False),
        scratch_types=[
            pltpu.VMEM((N_COLS,), jnp.float32),
            pltpu.VMEM((N_COLS,), jnp.float32),
            pltpu.VMEM((N_COLS,), jnp.float32),
            pltpu.VMEM((N_COLS,), jnp.float32),
            pltpu.VMEM((ROWS_PER_W * N_COLS,), jnp.int32),
            pltpu.VMEM((ROWS_PER_W * N_COLS,), jnp.int32),
            pltpu.VMEM((ROWS_PER_W * 256,), jnp.float32),
            pltpu.VMEM((16,), jnp.int32),
            pltpu.VMEM((16,), jnp.int32),
            pltpu.VMEM((16,), jnp.int32),
            pltpu.VMEM((16,), jnp.int32),
            pltpu.SemaphoreType.DMA,
            pltpu.SemaphoreType.DMA,
        ],
    )
    tau_packed = sc(outputs)

    out = pl.pallas_call(
        _tc_finish,
        out_shape=jax.ShapeDtypeStruct((1, 1), jnp.float32),
    )(outputs, t32, tau_packed)
    return out.reshape(())


# speedup vs baseline: 1.1955x; 1.0018x over previous
import functools

import jax
import jax.numpy as jnp
from jax import lax
from jax.experimental import pallas as pl
from jax.experimental.pallas import tpu as pltpu
from jax.experimental.pallas import tpu_sc as plsc

K = 256
N_ROWS = 128
N_COLS = 2048
NW = 32
ROWS_PER_W = N_ROWS // NW
NCHUNK = N_COLS // 16
INT_MIN32 = -2147483648


def _srl(a, n):
    return lax.shift_right_logical(a, jnp.full(a.shape, n, a.dtype))


def _sra(a, n):
    return lax.shift_right_arithmetic(a, jnp.full(a.shape, n, a.dtype))


def _sc_select(x_hbm, tau_hbm, x_v0, x_v1, x_v2, x_v3, keys_x, ckeys,
               hist4, o_v0, o_v1, o_v2, o_v3, sem_in, sem_out):
    wid = lax.axis_index("s") * 2 + lax.axis_index("c")
    base = wid * ROWS_PER_W
    xrows = [x_v0, x_v1, x_v2, x_v3]
    orows = [o_v0, o_v1, o_v2, o_v3]
    copies = [pltpu.async_copy(x_hbm.at[base + r], xrows[r], sem_in)
              for r in range(ROWS_PER_W)]
    for c in copies:
        c.wait()

    iota = lax.iota(jnp.int32, 16)
    lane16 = iota * 16
    ones = jnp.ones((16,), jnp.float32)
    zvec = jnp.zeros((16,), jnp.float32)
    zivec = jnp.zeros((16,), jnp.int32)

    def dyng(vec, lane_v):
        return vec.at[lane_v].get(mode="promise_in_bounds")

    def suffix(vec):
        return lax.rev(jnp.cumsum(lax.rev(vec, (0,))), (0,))

    def scan_hist(rem_v, hb):
        totv = zvec
        for c in range(16):
            totv = totv + plsc.load_gather(hist4, [hb + lane16 + c])
        sstv = suffix(totv)
        pcv = zivec + plsc.all_reduce_population_count(sstv >= rem_v)
        jstar = pcv - 1
        tot_j = dyng(totv, jstar)
        sst_j = dyng(sstv, jstar)
        above_vecs = sst_j - tot_j
        h = plsc.load_gather(hist4, [hb + jstar * 16 + iota])
        ssv = suffix(h)
        pc2 = zivec + plsc.all_reduce_population_count(
            (above_vecs + ssv) >= rem_v)
        lstar = pc2 - 1
        ss_l = dyng(ssv, lstar)
        h_l = dyng(h, lstar)
        bstar = jstar * 16 + lstar
        above = above_vecs + ss_l - h_l
        return bstar, above, h_l

    def zero_hists():
        def zb(i, c):
            for u in range(4):
                hist4[pl.ds((i * 4 + u) * 16, 16)] = zvec
            return c
        lax.fori_loop(0, 16, zb, jnp.int32(0))

    def key_of(xv):
        ib = lax.bitcast_convert_type(xv, jnp.int32)
        sgn = _sra(ib, 31)
        return ib ^ ((sgn & 0x7FFFFFFF) | INT_MIN32)

    NR = ROWS_PER_W
    U = 2

    zero_hists()

    def body0(i, c):
        for u in range(U):
            off = i * (16 * U) + u * 16
            for r in range(NR):
                uk = key_of(xrows[r][pl.ds(off, 16)])
                keys_x[pl.ds(r * N_COLS + off, 16)] = uk
                plsc.addupdate_scatter(hist4, [r * 256 + _srl(uk, 24)],
                                       ones)
        return c
    lax.fori_loop(0, NCHUNK // U, body0, jnp.int32(0))

    rems = []
    b0s = []
    n1s = []
    for r in range(NR):
        b0, above0, n1f = scan_hist(zvec + jnp.float32(K), r * 256)
        rems.append(zvec + jnp.float32(K) - above0)
        b0s.append(b0)
        n1s.append(n1f.astype(jnp.int32))

    def bodyc(i, offs):
        offs = list(offs)
        for u in range(U):
            off = i * (16 * U) + u * 16
            for r in range(NR):
                uk = keys_x[pl.ds(r * N_COLS + off, 16)]
                msk = _srl(uk, 24) == b0s[r]
                mi = jnp.where(msk, jnp.int32(1), jnp.int32(0))
                excl = jnp.cumsum(mi) - mi
                plsc.store_scatter(ckeys, [r * N_COLS + offs[r] + excl],
                                   uk, mask=msk)
                offs[r] = offs[r] + plsc.all_reduce_population_count(msk)
        return tuple(offs)
    lax.fori_loop(0, NCHUNK // U, bodyc, tuple(zivec for _ in range(NR)))

    nch1 = jnp.max(_srl(jnp.maximum(jnp.maximum(n1s[0], n1s[1]),
                                    jnp.maximum(n1s[2], n1s[3])) + 15, 4))

    b1s = b2s = b3s = None
    for l in range(1, 4):
        zero_hists()

        def bodyl(i, c, _l=l, _b1=b1s, _b2=b2s):
            for r in range(NR):
                uk = ckeys[pl.ds(r * N_COLS + i * 16, 16)]
                msk = (i * 16 + iota) < n1s[r]
                if _l >= 2:
                    msk = msk & ((_srl(uk, 16) & 0xFF) == _b1[r])
                if _l >= 3:
                    msk = msk & ((_srl(uk, 8) & 0xFF) == _b2[r])
                binv = _srl(uk, 24 - 8 * _l) & 0xFF
                plsc.addupdate_scatter(hist4, [r * 256 + binv], ones,
                                       mask=msk)
            return c
        lax.fori_loop(0, nch1, bodyl, jnp.int32(0))

        bls = []
        for r in range(NR):
            bl, above_l, _hl = scan_hist(rems[r], r * 256)
            rems[r] = rems[r] - above_l
            bls.append(bl)
        if l == 1:
            b1s = bls
        elif l == 2:
            b2s = bls
        else:
            b3s = bls

    out_copies = []
    for r in range(NR):
        prefix = (jnp.left_shift(b0s[r], 24) | jnp.left_shift(b1s[r], 16)
                  | jnp.left_shift(b2s[r], 8) | b3s[r])
        orows[r][...] = prefix
        out_copies.append(
            pltpu.async_copy(orows[r], tau_hbm.at[base + r], sem_out))
    for c in out_copies:
        c.wait()


def _tc_pre(x_ref, t_ref, m_ref, v_ref, tu_ref):
    x = x_ref[:]
    t = t_ref[:]
    m_ref[:, :] = jnp.max(x, axis=1, keepdims=True)
    col = jax.lax.broadcasted_iota(jnp.int32, (N_ROWS, N_COLS), 1)
    at_t = col == t
    v_ref[:, :] = jnp.sum(jnp.where(at_t, x, 0.0), axis=1, keepdims=True)
    ub = jax.lax.bitcast_convert_type(x, jnp.uint32)
    ukey = jnp.where(ub >= jnp.uint32(0x80000000), ~ub,
                     ub | jnp.uint32(0x80000000))
    ukey_i = jax.lax.bitcast_convert_type(ukey, jnp.int32)
    tu_ref[:, :] = jnp.sum(jnp.where(at_t, ukey_i, 0), axis=1,
                           keepdims=True)


def _tc_finish(x_ref, t_ref, p_ref, m_ref, v_ref, tu_ref, out_ref):
    x = x_ref[:]
    t = t_ref[:]
    p_i = p_ref[:][:, :1]
    p = jax.lax.bitcast_convert_type(p_i, jnp.uint32)
    m = m_ref[:]
    v = v_ref[:]
    tu = jax.lax.bitcast_convert_type(tu_ref[:], jnp.uint32)

    ub = jax.lax.bitcast_convert_type(x, jnp.uint32)
    ukey = jnp.where(ub >= jnp.uint32(0x80000000), ~ub,
                     ub | jnp.uint32(0x80000000))

    ub_tau = jnp.where(p >= jnp.uint32(0x80000000),
                       p ^ jnp.uint32(0x80000000), ~p)
    tau = jax.lax.bitcast_convert_type(ub_tau, jnp.float32)

    gt = ukey > p
    c_gt = jnp.sum(gt.astype(jnp.int32), axis=1, keepdims=True)
    e = jnp.exp(x - m)
    s_above = jnp.sum(jnp.where(gt, e, 0.0), axis=1, keepdims=True)
    S = s_above + (K - c_gt).astype(jnp.float32) * jnp.exp(tau - m)

    col = jax.lax.broadcasted_iota(jnp.int32, (N_ROWS, N_COLS), 1)
    eq_before = jnp.sum(((col < t) & (ukey == p)).astype(jnp.int32),
                        axis=1, keepdims=True)
    in_topk = (tu > p) | ((tu == p) & (c_gt + eq_before < K))
    inf = in_topk.astype(jnp.float32)

    contrib = v - m - jnp.log(S)
    total = jnp.sum(inf * contrib)
    count = jnp.sum(inf)
    out_ref[:, :] = jnp.full((1, 1), -(total / count), dtype=jnp.float32)


def kernel(outputs, targets):
    t32 = targets.astype(jnp.int32).reshape(N_ROWS, 1)

    sc = pl.kernel(
        _sc_select,
        out_type=jax.ShapeDtypeStruct((N_ROWS, 16), jnp.int32),
        mesh=plsc.VectorSubcoreMesh(core_axis_name="c", subcore_axis_name="s"),
        compiler_params=pltpu.CompilerParams(needs_layout_passes=False),
        scratch_types=[
            pltpu.VMEM((N_COLS,), jnp.float32),
            pltpu.VMEM((N_COLS,), jnp.float32),
            pltpu.VMEM((N_COLS,), jnp.float32),
            pltpu.VMEM((N_COLS,), jnp.float32),
            pltpu.VMEM((ROWS_PER_W * N_COLS,), jnp.int32),
            pltpu.VMEM((ROWS_PER_W * N_COLS,), jnp.int32),
            pltpu.VMEM((ROWS_PER_W * 256,), jnp.float32),
            pltpu.VMEM((16,), jnp.int32),
            pltpu.VMEM((16,), jnp.int32),
            pltpu.VMEM((16,), jnp.int32),
            pltpu.VMEM((16,), jnp.int32),
            pltpu.SemaphoreType.DMA,
            pltpu.SemaphoreType.DMA,
        ],
    )
    tau_packed = sc(outputs)

    m, v, tu = pl.pallas_call(
        _tc_pre,
        out_shape=[
            jax.ShapeDtypeStruct((N_ROWS, 1), jnp.float32),
            jax.ShapeDtypeStruct((N_ROWS, 1), jnp.float32),
            jax.ShapeDtypeStruct((N_ROWS, 1), jnp.int32),
        ],
    )(outputs, t32)

    out = pl.pallas_call(
        _tc_finish,
        out_shape=jax.ShapeDtypeStruct((1, 1), jnp.float32),
    )(outputs, t32, tau_packed, m, v, tu)
    return out.reshape(())


# speedup vs baseline: 1.1964x; 1.0008x over previous
import jax
import jax.numpy as jnp
from jax import lax
from jax.experimental import pallas as pl
from jax.experimental.pallas import tpu as pltpu
from jax.experimental.pallas import tpu_sc as plsc

K = 256
N_ROWS = 128
N_COLS = 2048
NW = 32
ROWS_PER_W = N_ROWS // NW
NCHUNK = N_COLS // 16
INT_MIN32 = -2147483648


def _srl(a, n):
    return lax.shift_right_logical(a, jnp.full(a.shape, n, a.dtype))


def _sra(a, n):
    return lax.shift_right_arithmetic(a, jnp.full(a.shape, n, a.dtype))


def _sc_select(x_hbm, tau_hbm, x_v0, x_v1, x_v2, x_v3, keys_x, ckeys,
               hist4, o_v0, o_v1, o_v2, o_v3, sem_in, sem_out):
    wid = lax.axis_index("s") * 2 + lax.axis_index("c")
    base = wid * ROWS_PER_W
    xrows = [x_v0, x_v1, x_v2, x_v3]
    orows = [o_v0, o_v1, o_v2, o_v3]
    copies = [pltpu.async_copy(x_hbm.at[base + r], xrows[r], sem_in)
              for r in range(ROWS_PER_W)]
    for c in copies:
        c.wait()

    iota = lax.iota(jnp.int32, 16)
    lane16 = iota * 16
    ones = jnp.ones((16,), jnp.float32)
    zvec = jnp.zeros((16,), jnp.float32)
    zivec = jnp.zeros((16,), jnp.int32)

    def dyng(vec, lane_v):
        return vec.at[lane_v].get(mode="promise_in_bounds")

    def suffix(vec):
        return lax.rev(jnp.cumsum(lax.rev(vec, (0,))), (0,))

    def scan_hist(rem_v, hb):
        totv = zvec
        for c in range(16):
            totv = totv + plsc.load_gather(hist4, [hb + lane16 + c])
        sstv = suffix(totv)
        pcv = zivec + plsc.all_reduce_population_count(sstv >= rem_v)
        jstar = pcv - 1
        tot_j = dyng(totv, jstar)
        sst_j = dyng(sstv, jstar)
        above_vecs = sst_j - tot_j
        h = plsc.load_gather(hist4, [hb + jstar * 16 + iota])
        ssv = suffix(h)
        pc2 = zivec + plsc.all_reduce_population_count(
            (above_vecs + ssv) >= rem_v)
        lstar = pc2 - 1
        ss_l = dyng(ssv, lstar)
        h_l = dyng(h, lstar)
        bstar = jstar * 16 + lstar
        above = above_vecs + ss_l - h_l
        return bstar, above, h_l

    def zero_hists():
        def zb(i, c):
            for u in range(4):
                hist4[pl.ds((i * 4 + u) * 16, 16)] = zvec
            return c
        lax.fori_loop(0, 16, zb, jnp.int32(0))

    def key_of(xv):
        ib = lax.bitcast_convert_type(xv, jnp.int32)
        sgn = _sra(ib, 31)
        return ib ^ ((sgn & 0x7FFFFFFF) | INT_MIN32)

    NR = ROWS_PER_W
    U = 2

    zero_hists()

    def body0(i, c):
        for u in range(U):
            off = i * (16 * U) + u * 16
            for r in range(NR):
                uk = key_of(xrows[r][pl.ds(off, 16)])
                keys_x[pl.ds(r * N_COLS + off, 16)] = uk
                plsc.addupdate_scatter(hist4, [r * 256 + _srl(uk, 24)],
                                       ones)
        return c
    lax.fori_loop(0, NCHUNK // U, body0, jnp.int32(0))

    rems = []
    b0s = []
    n1s = []
    for r in range(NR):
        b0, above0, n1f = scan_hist(zvec + jnp.float32(K), r * 256)
        rems.append(zvec + jnp.float32(K) - above0)
        b0s.append(b0)
        n1s.append(n1f.astype(jnp.int32))

    def bodyc(i, offs):
        offs = list(offs)
        for u in range(U):
            off = i * (16 * U) + u * 16
            for r in range(NR):
                uk = keys_x[pl.ds(r * N_COLS + off, 16)]
                msk = _srl(uk, 24) == b0s[r]
                mi = jnp.where(msk, jnp.int32(1), jnp.int32(0))
                excl = jnp.cumsum(mi) - mi
                plsc.store_scatter(ckeys, [r * N_COLS + offs[r] + excl],
                                   uk, mask=msk)
                offs[r] = offs[r] + plsc.all_reduce_population_count(msk)
        return tuple(offs)
    lax.fori_loop(0, NCHUNK // U, bodyc, tuple(zivec for _ in range(NR)))

    nch1 = jnp.max(_srl(jnp.maximum(jnp.maximum(n1s[0], n1s[1]),
                                    jnp.maximum(n1s[2], n1s[3])) + 15, 4))

    b1s = b2s = b3s = None
    for l in range(1, 4):
        zero_hists()

        def bodyl(i, c, _l=l, _b1=b1s, _b2=b2s):
            for r in range(NR):
                uk = ckeys[pl.ds(r * N_COLS + i * 16, 16)]
                msk = (i * 16 + iota) < n1s[r]
                if _l >= 2:
                    msk = msk & ((_srl(uk, 16) & 0xFF) == _b1[r])
                if _l >= 3:
                    msk = msk & ((_srl(uk, 8) & 0xFF) == _b2[r])
                binv = _srl(uk, 24 - 8 * _l) & 0xFF
                plsc.addupdate_scatter(hist4, [r * 256 + binv], ones,
                                       mask=msk)
            return c
        lax.fori_loop(0, nch1, bodyl, jnp.int32(0))

        bls = []
        for r in range(NR):
            bl, above_l, _hl = scan_hist(rems[r], r * 256)
            rems[r] = rems[r] - above_l
            bls.append(bl)
        if l == 1:
            b1s = bls
        elif l == 2:
            b2s = bls
        else:
            b3s = bls

    out_copies = []
    for r in range(NR):
        prefix = (jnp.left_shift(b0s[r], 24) | jnp.left_shift(b1s[r], 16)
                  | jnp.left_shift(b2s[r], 8) | b3s[r])
        orows[r][...] = prefix
        out_copies.append(
            pltpu.async_copy(orows[r], tau_hbm.at[base + r], sem_out))
    for c in out_copies:
        c.wait()


def _tc_pre(x_ref, t_ref, m_ref, v_ref, tu_ref):
    x = x_ref[:]
    t = t_ref[:]
    m_ref[:, :] = jnp.max(x, axis=1, keepdims=True)
    col = jax.lax.broadcasted_iota(jnp.int32, (N_ROWS, N_COLS), 1)
    at_t = col == t
    v_ref[:, :] = jnp.sum(jnp.where(at_t, x, 0.0), axis=1, keepdims=True)
    ub = jax.lax.bitcast_convert_type(x, jnp.uint32)
    ukey = jnp.where(ub >= jnp.uint32(0x80000000), ~ub,
                     ub | jnp.uint32(0x80000000))
    ukey_i = jax.lax.bitcast_convert_type(ukey, jnp.int32)
    tu_ref[:, :] = jnp.sum(jnp.where(at_t, ukey_i, 0), axis=1,
                           keepdims=True)


def _tc_finish(x_ref, t_ref, p_ref, m_ref, v_ref, tu_ref, out_ref):
    x = x_ref[:]
    t = t_ref[:]
    p_i = p_ref[:][:, :1]
    p = jax.lax.bitcast_convert_type(p_i, jnp.uint32)
    m = m_ref[:]
    v = v_ref[:]
    tu = jax.lax.bitcast_convert_type(tu_ref[:], jnp.uint32)

    ub = jax.lax.bitcast_convert_type(x, jnp.uint32)
    ukey = jnp.where(ub >= jnp.uint32(0x80000000), ~ub,
                     ub | jnp.uint32(0x80000000))

    ub_tau = jnp.where(p >= jnp.uint32(0x80000000),
                       p ^ jnp.uint32(0x80000000), ~p)
    tau = jax.lax.bitcast_convert_type(ub_tau, jnp.float32)

    gt = ukey > p
    c_gt = jnp.sum(gt.astype(jnp.int32), axis=1, keepdims=True)
    e = jnp.exp(x - m)
    s_above = jnp.sum(jnp.where(gt, e, 0.0), axis=1, keepdims=True)
    S = s_above + (K - c_gt).astype(jnp.float32) * jnp.exp(tau - m)

    col = jax.lax.broadcasted_iota(jnp.int32, (N_ROWS, N_COLS), 1)
    eq_before = jnp.sum(((col < t) & (ukey == p)).astype(jnp.int32),
                        axis=1, keepdims=True)
    in_topk = (tu > p) | ((tu == p) & (c_gt + eq_before < K))
    inf = in_topk.astype(jnp.float32)

    contrib = v - m - jnp.log(S)
    total = jnp.sum(inf * contrib)
    count = jnp.sum(inf)
    out_ref[:, :] = jnp.full((1, 1), -(total / count), dtype=jnp.float32)


def kernel(outputs, targets):
    t32 = targets.astype(jnp.int32).reshape(N_ROWS, 1)

    sc = pl.kernel(
        _sc_select,
        out_type=jax.ShapeDtypeStruct((N_ROWS, 16), jnp.int32),
        mesh=plsc.VectorSubcoreMesh(core_axis_name="c", subcore_axis_name="s"),
        compiler_params=pltpu.CompilerParams(needs_layout_passes=False),
        scratch_types=[
            pltpu.VMEM((N_COLS,), jnp.float32),
            pltpu.VMEM((N_COLS,), jnp.float32),
            pltpu.VMEM((N_COLS,), jnp.float32),
            pltpu.VMEM((N_COLS,), jnp.float32),
            pltpu.VMEM((ROWS_PER_W * N_COLS,), jnp.int32),
            pltpu.VMEM((ROWS_PER_W * N_COLS,), jnp.int32),
            pltpu.VMEM((ROWS_PER_W * 256,), jnp.float32),
            pltpu.VMEM((16,), jnp.int32),
            pltpu.VMEM((16,), jnp.int32),
            pltpu.VMEM((16,), jnp.int32),
            pltpu.VMEM((16,), jnp.int32),
            pltpu.SemaphoreType.DMA,
            pltpu.SemaphoreType.DMA,
        ],
    )
    tau_packed = sc(outputs)

    m, v, tu = pl.pallas_call(
        _tc_pre,
        out_shape=[
            jax.ShapeDtypeStruct((N_ROWS, 1), jnp.float32),
            jax.ShapeDtypeStruct((N_ROWS, 1), jnp.float32),
            jax.ShapeDtypeStruct((N_ROWS, 1), jnp.int32),
        ],
    )(outputs, t32)

    out = pl.pallas_call(
        _tc_finish,
        out_shape=jax.ShapeDtypeStruct((1, 1), jnp.float32),
    )(outputs, t32, tau_packed, m, v, tu)
    return out.reshape(())
